# Initial kernel scaffold; baseline (speedup 1.0000x reference)
import jax
import jax.numpy as jnp
from jax.experimental import pallas as pl


def kernel(hidden_states, weight):
    raise NotImplementedError("write your pallas kernel here")


# speedup vs baseline: 1.3651x; 1.3651x over previous
import functools

import jax
import jax.numpy as jnp
from jax.experimental import pallas as pl
from jax.experimental.pallas import tpu as pltpu

N_EXPERTS = 64
K = 8
ALPHA = 0.01


def _gate_kernel(hs_ref, wt_ref, tw_ref, ti_ref, aux_ref, pi_acc, cnt_acc,
                 *, num_steps, total_tokens):
    step = pl.program_id(0)

    @pl.when(step == 0)
    def _init():
        pi_acc[...] = jnp.zeros_like(pi_acc)
        cnt_acc[...] = jnp.zeros_like(cnt_acc)

    x = hs_ref[...]
    logits = jnp.dot(x, wt_ref[...], preferred_element_type=jnp.float32)
    m = jnp.max(logits, axis=1, keepdims=True)
    e = jnp.exp(logits - m)
    p = e / jnp.sum(e, axis=1, keepdims=True)

    pi_acc[0, :] = pi_acc[0, :] + jnp.sum(p, axis=0)

    iota = jax.lax.broadcasted_iota(jnp.int32, p.shape, 1)
    work = p
    ws = []
    idxs = []
    cnt = cnt_acc[0, :]
    for _ in range(K):
        mx = jnp.max(work, axis=1, keepdims=True)
        eq = work == mx
        sel = jnp.min(jnp.where(eq, iota, N_EXPERTS), axis=1)
        onehot = iota == sel[:, None]
        ws.append(mx[:, 0])
        idxs.append(sel)
        cnt = cnt + jnp.sum(onehot.astype(jnp.float32), axis=0)
        work = jnp.where(onehot, -1.0, work)
    cnt_acc[0, :] = cnt

    tw_ref[...] = jnp.stack(ws, axis=1)
    ti_ref[...] = jnp.stack(idxs, axis=1)

    @pl.when(step == num_steps - 1)
    def _finish():
        scale = ALPHA * N_EXPERTS / (float(total_tokens) * float(total_tokens) * K)
        aux_ref[0, 0] = jnp.sum(pi_acc[0, :] * cnt_acc[0, :]) * scale


@jax.jit
def _gate(hs, wt):
    t, h = hs.shape
    block = 2048
    num_steps = t // block
    kfn = functools.partial(_gate_kernel, num_steps=num_steps, total_tokens=t)
    tw, ti, aux = pl.pallas_call(
        kfn,
        grid=(num_steps,),
        in_specs=[
            pl.BlockSpec((block, h), lambda i: (i, 0)),
            pl.BlockSpec((h, N_EXPERTS), lambda i: (0, 0)),
        ],
        out_specs=[
            pl.BlockSpec((block, K), lambda i: (i, 0)),
            pl.BlockSpec((block, K), lambda i: (i, 0)),
            pl.BlockSpec(memory_space=pltpu.SMEM),
        ],
        out_shape=[
            jax.ShapeDtypeStruct((t, K), jnp.float32),
            jax.ShapeDtypeStruct((t, K), jnp.int32),
            jax.ShapeDtypeStruct((1, 1), jnp.float32),
        ],
        scratch_shapes=[
            pltpu.VMEM((1, N_EXPERTS), jnp.float32),
            pltpu.VMEM((1, N_EXPERTS), jnp.float32),
        ],
        compiler_params=pltpu.CompilerParams(
            dimension_semantics=("arbitrary",),
        ),
    )(hs, wt)
    return tw, ti, aux[0, 0]


def kernel(hidden_states, weight):
    bsz, seq_len, h = hidden_states.shape
    hs = hidden_states.reshape(-1, h)
    tw, ti, aux = _gate(hs, weight.T)
    return tw, ti, aux


# speedup vs baseline: 2.8589x; 2.0942x over previous
import functools

import jax
import jax.numpy as jnp
from jax.experimental import pallas as pl
from jax.experimental.pallas import tpu as pltpu

N_EXPERTS = 64
K = 8
ALPHA = 0.01


def _gate_kernel(hs_ref, w_ref, tw_ref, ti_ref, aux_ref, ps_acc, cnt_acc,
                 *, num_steps, total_tokens):
    step = pl.program_id(0)

    @pl.when(step == 0)
    def _init():
        ps_acc[...] = jnp.zeros_like(ps_acc)
        cnt_acc[...] = jnp.zeros_like(cnt_acc)

    x = hs_ref[...]
    w = w_ref[...]
    lt = jax.lax.dot_general(w, x, (((1,), (1,)), ((), ())),
                             preferred_element_type=jnp.float32)
    m = jnp.max(lt, axis=0, keepdims=True)
    ex = jnp.exp(lt - m)
    s = jnp.sum(ex, axis=0, keepdims=True)
    p = ex / s

    ps_acc[...] += p

    iota = jax.lax.broadcasted_iota(jnp.int32, p.shape, 0)
    work = p
    ws = []
    idxs = []
    for _ in range(K):
        mx = jnp.max(work, axis=0, keepdims=True)
        sel = jnp.min(jnp.where(work == mx, iota, N_EXPERTS),
                      axis=0, keepdims=True)
        work = jnp.where(iota == sel, -1.0, work)
        ws.append(mx)
        idxs.append(sel)

    cnt_acc[...] += (work < 0).astype(jnp.float32)

    tw_ref[...] = jnp.concatenate(ws, axis=0).T
    ti_ref[...] = jnp.concatenate(idxs, axis=0).T

    @pl.when(step == num_steps - 1)
    def _finish():
        pi = jnp.sum(ps_acc[...], axis=1)
        c = jnp.sum(cnt_acc[...], axis=1)
        scale = ALPHA * N_EXPERTS / (float(total_tokens) * float(total_tokens) * K)
        aux_ref[0, 0] = jnp.sum(pi * c) * scale


@jax.jit
def _gate(hs, w):
    t, h = hs.shape
    block = 2048
    num_steps = t // block
    kfn = functools.partial(_gate_kernel, num_steps=num_steps, total_tokens=t)
    tw, ti, aux = pl.pallas_call(
        kfn,
        grid=(num_steps,),
        in_specs=[
            pl.BlockSpec((block, h), lambda i: (i, 0)),
            pl.BlockSpec((N_EXPERTS, h), lambda i: (0, 0)),
        ],
        out_specs=[
            pl.BlockSpec((block, K), lambda i: (i, 0)),
            pl.BlockSpec((block, K), lambda i: (i, 0)),
            pl.BlockSpec(memory_space=pltpu.SMEM),
        ],
        out_shape=[
            jax.ShapeDtypeStruct((t, K), jnp.float32),
            jax.ShapeDtypeStruct((t, K), jnp.int32),
            jax.ShapeDtypeStruct((1, 1), jnp.float32),
        ],
        scratch_shapes=[
            pltpu.VMEM((N_EXPERTS, block), jnp.float32),
            pltpu.VMEM((N_EXPERTS, block), jnp.float32),
        ],
        compiler_params=pltpu.CompilerParams(
            dimension_semantics=("arbitrary",),
        ),
    )(hs, w)
    return tw, ti, aux[0, 0]


def kernel(hidden_states, weight):
    bsz, seq_len, h = hidden_states.shape
    hs = hidden_states.reshape(-1, h)
    tw, ti, aux = _gate(hs, weight)
    return tw, ti, aux


# speedup vs baseline: 3.0340x; 1.0613x over previous
import functools

import jax
import jax.numpy as jnp
from jax.experimental import pallas as pl
from jax.experimental.pallas import tpu as pltpu

N_EXPERTS = 64
K = 8
ALPHA = 0.01


def _gate_kernel(hs_ref, w_ref, tw_ref, ti_ref, aux_ref, ps_acc, cnt_acc,
                 *, num_steps, total_tokens):
    step = pl.program_id(0)

    @pl.when(step == 0)
    def _init():
        ps_acc[...] = jnp.zeros_like(ps_acc)
        cnt_acc[...] = jnp.zeros_like(cnt_acc)

    x = hs_ref[...]
    w = w_ref[...]
    lt = jax.lax.dot_general(w, x, (((1,), (1,)), ((), ())),
                             preferred_element_type=jnp.float32)
    m = jnp.max(lt, axis=0, keepdims=True)
    ex = jnp.exp(lt - m)
    s = jnp.sum(ex, axis=0, keepdims=True)
    p = ex / s

    ps_acc[...] += p

    iota = jax.lax.broadcasted_iota(jnp.int32, p.shape, 0)
    work = p
    ws = []
    idxs = []
    for _ in range(K):
        mx = jnp.max(work, axis=0, keepdims=True)
        sel = jnp.min(jnp.where(work == mx, iota, N_EXPERTS),
                      axis=0, keepdims=True)
        work = jnp.where(iota == sel, -1.0, work)
        ws.append(mx)
        idxs.append(sel)

    cnt_acc[...] += (work < 0).astype(jnp.float32)

    tw_ref[...] = jnp.concatenate(ws, axis=0).T
    ti_ref[...] = jnp.concatenate(idxs, axis=0).T

    @pl.when(step == num_steps - 1)
    def _finish():
        pi = jnp.sum(ps_acc[...], axis=1)
        c = jnp.sum(cnt_acc[...], axis=1)
        scale = ALPHA * N_EXPERTS / (float(total_tokens) * float(total_tokens) * K)
        aux_ref[0, 0] = jnp.sum(pi * c) * scale


@jax.jit
def _gate(hs, w):
    t, h = hs.shape
    block = 4096
    num_steps = t // block
    kfn = functools.partial(_gate_kernel, num_steps=num_steps, total_tokens=t)
    tw, ti, aux = pl.pallas_call(
        kfn,
        grid=(num_steps,),
        in_specs=[
            pl.BlockSpec((block, h), lambda i: (i, 0)),
            pl.BlockSpec((N_EXPERTS, h), lambda i: (0, 0)),
        ],
        out_specs=[
            pl.BlockSpec((block, K), lambda i: (i, 0)),
            pl.BlockSpec((block, K), lambda i: (i, 0)),
            pl.BlockSpec(memory_space=pltpu.SMEM),
        ],
        out_shape=[
            jax.ShapeDtypeStruct((t, K), jnp.float32),
            jax.ShapeDtypeStruct((t, K), jnp.int32),
            jax.ShapeDtypeStruct((1, 1), jnp.float32),
        ],
        scratch_shapes=[
            pltpu.VMEM((N_EXPERTS, block), jnp.float32),
            pltpu.VMEM((N_EXPERTS, block), jnp.float32),
        ],
        compiler_params=pltpu.CompilerParams(
            dimension_semantics=("arbitrary",),
        ),
    )(hs, w)
    return tw, ti, aux[0, 0]


def kernel(hidden_states, weight):
    bsz, seq_len, h = hidden_states.shape
    hs = hidden_states.reshape(-1, h)
    tw, ti, aux = _gate(hs, weight)
    return tw, ti, aux


# speedup vs baseline: 3.0345x; 1.0002x over previous
import functools

import jax
import jax.numpy as jnp
from jax.experimental import pallas as pl
from jax.experimental.pallas import tpu as pltpu

N_EXPERTS = 64
K = 8
ALPHA = 0.01


def _route_block(x, w, ps_acc, cnt_acc):
    lt = jax.lax.dot_general(w, x, (((1,), (1,)), ((), ())),
                             preferred_element_type=jnp.float32)
    m = jnp.max(lt, axis=0, keepdims=True)
    ex = jnp.exp(lt - m)
    s = jnp.sum(ex, axis=0, keepdims=True)
    p = ex / s

    ps_acc[...] += p

    iota = jax.lax.broadcasted_iota(jnp.int32, p.shape, 0)
    work = p
    ws = []
    idxs = []
    for _ in range(K):
        mx = jnp.max(work, axis=0, keepdims=True)
        sel = jnp.min(jnp.where(work == mx, iota, N_EXPERTS),
                      axis=0, keepdims=True)
        work = jnp.where(iota == sel, -1.0, work)
        ws.append(mx)
        idxs.append(sel)

    cnt_acc[...] += (work < 0).astype(jnp.float32)

    return jnp.concatenate(ws, axis=0).T, jnp.concatenate(idxs, axis=0).T


def _gate_kernel(hs1_ref, hs2_ref, w_ref, tw_ref, ti_ref, aux_ref,
                 ps_acc, cnt_acc, *, num_steps, total_tokens, block):
    step = pl.program_id(0)

    @pl.when(step == 0)
    def _init():
        ps_acc[...] = jnp.zeros_like(ps_acc)
        cnt_acc[...] = jnp.zeros_like(cnt_acc)

    w = w_ref[...]
    tw1, ti1 = _route_block(hs1_ref[...], w, ps_acc, cnt_acc)
    tw2, ti2 = _route_block(hs2_ref[...], w, ps_acc, cnt_acc)
    tw_ref[0:block, :] = tw1
    tw_ref[block:2 * block, :] = tw2
    ti_ref[0:block, :] = ti1
    ti_ref[block:2 * block, :] = ti2

    @pl.when(step == num_steps - 1)
    def _finish():
        pi = jnp.sum(ps_acc[...], axis=1)
        c = jnp.sum(cnt_acc[...], axis=1)
        scale = ALPHA * N_EXPERTS / (float(total_tokens) * float(total_tokens) * K)
        aux_ref[0, 0] = jnp.sum(pi * c) * scale


@jax.jit
def _gate(hs, w):
    t, h = hs.shape
    block = 2048
    num_steps = t // (2 * block)
    kfn = functools.partial(_gate_kernel, num_steps=num_steps,
                            total_tokens=t, block=block)
    tw, ti, aux = pl.pallas_call(
        kfn,
        grid=(num_steps,),
        in_specs=[
            pl.BlockSpec((block, h), lambda i: (2 * i, 0)),
            pl.BlockSpec((block, h), lambda i: (2 * i + 1, 0)),
            pl.BlockSpec((N_EXPERTS, h), lambda i: (0, 0)),
        ],
        out_specs=[
            pl.BlockSpec((2 * block, K), lambda i: (i, 0)),
            pl.BlockSpec((2 * block, K), lambda i: (i, 0)),
            pl.BlockSpec(memory_space=pltpu.SMEM),
        ],
        out_shape=[
            jax.ShapeDtypeStruct((t, K), jnp.float32),
            jax.ShapeDtypeStruct((t, K), jnp.int32),
            jax.ShapeDtypeStruct((1, 1), jnp.float32),
        ],
        scratch_shapes=[
            pltpu.VMEM((N_EXPERTS, block), jnp.float32),
            pltpu.VMEM((N_EXPERTS, block), jnp.float32),
        ],
        compiler_params=pltpu.CompilerParams(
            dimension_semantics=("arbitrary",),
        ),
    )(hs, hs, w)
    return tw, ti, aux[0, 0]


def kernel(hidden_states, weight):
    bsz, seq_len, h = hidden_states.shape
    hs = hidden_states.reshape(-1, h)
    tw, ti, aux = _gate(hs, weight)
    return tw, ti, aux
